# baseline (speedup 1.0000x reference)
import functools

import jax
import jax.numpy as jnp
from jax import lax
from jax.experimental import pallas as pl
from jax.experimental.pallas import tpu as pltpu
from jax.experimental.pallas import tpu_sc as plsc

EMBED = 128
ROWS = 17
GROUPS = 50
GPAD = 56
DIM0 = 16384
NUM_WORKERS = 32
D0PW = DIM0 // NUM_WORKERS
N0 = 1
NCHUNK = D0PW // N0


def _lookup(idx_hbm, table_hbm, out_hbm, idx_v, table_sh, s0, s1, s2, s3,
            gsem, wsem):
  sid = lax.axis_index("s")
  wid = sid * 2 + lax.axis_index("c")
  base = wid * D0PW

  pl.when(sid == 0)(lambda: pltpu.sync_copy(table_hbm, table_sh))
  pltpu.sync_copy(idx_hbm.at[pl.ds(base, D0PW)], idx_v)
  plsc.subcore_barrier()

  bufs = (s0, s1, s2, s3)

  def start_gather(b, i):
    pltpu.async_copy(
        table_sh.at[idx_v.at[i]], bufs[b].at[pl.ds(0, GROUPS)], gsem)

  def wait_gather(b):
    pltpu.make_async_copy(
        table_sh.at[idx_v.at[0]], bufs[b].at[pl.ds(0, GROUPS)], gsem).wait()

  def start_write(b, i):
    pltpu.async_copy(bufs[b], out_hbm.at[base + i], wsem)

  def drain_write(b):
    pltpu.make_async_copy(bufs[b], out_hbm.at[0], wsem).wait()

  for j in range(3):
    start_gather(j, j)

  def quad_body(o, _):
    for j in range(4):
      i = 4 * o + j
      wait_gather(j)
      start_write(j, i)
      if j == 0:
        pl.when(i >= 1)(lambda: drain_write(0))
      else:
        drain_write(j - 1)
      nb = (j + 3) % 4
      pl.when(i + 3 < NCHUNK)(lambda i=i, nb=nb: start_gather(nb, i + 3))
    return 0

  lax.fori_loop(0, NCHUNK // 4, quad_body, 0)
  drain_write(3)


def kernel(group_idx, table):
  mesh = plsc.VectorSubcoreMesh(core_axis_name="c", subcore_axis_name="s")
  run = functools.partial(
      pl.kernel,
      out_type=jax.ShapeDtypeStruct((DIM0, GPAD, EMBED), jnp.float32),
      mesh=mesh,
      compiler_params=pltpu.CompilerParams(
          needs_layout_passes=False, use_tc_tiling_on_sc=True),
      scratch_types=[
          pltpu.VMEM((D0PW, GROUPS), jnp.int32),
          pltpu.VMEM_SHARED((ROWS, EMBED), jnp.float32),
          pltpu.VMEM((GPAD, EMBED), jnp.float32),
          pltpu.VMEM((GPAD, EMBED), jnp.float32),
          pltpu.VMEM((GPAD, EMBED), jnp.float32),
          pltpu.VMEM((GPAD, EMBED), jnp.float32),
          pltpu.SemaphoreType.DMA,
          pltpu.SemaphoreType.DMA,
      ],
  )(_lookup)
  return run(group_idx, table)[:, :GROUPS, :]


# speedup vs baseline: 3.0382x; 3.0382x over previous
import functools

import jax
import jax.numpy as jnp
from jax import lax
from jax.experimental import pallas as pl
from jax.experimental.pallas import tpu as pltpu
from jax.experimental.pallas import tpu_sc as plsc

EMBED = 128
ROWS = 17
GROUPS = 50
DIM0 = 16384
NUM_WORKERS = 32
CPW = DIM0 // NUM_WORKERS
CHUNK = 128
QPJ = CPW // CHUNK


def _lookup(idx_hbm, table_hbm, out_hbm, idx_v, table_sh, s0, s1, s2, s3,
            gsem, wsem):
  sid = lax.axis_index("s")
  wid = sid * 2 + lax.axis_index("c")
  col0 = wid * CPW

  pl.when(sid == 0)(lambda: pltpu.sync_copy(table_hbm, table_sh))
  pltpu.sync_copy(idx_hbm.at[pl.ds(0, GROUPS), pl.ds(col0, CPW)], idx_v)
  plsc.subcore_barrier()

  bufs = (s0, s1, s2, s3)

  def start_gather(b, j, q):
    pltpu.async_copy(
        table_sh.at[idx_v.at[j, pl.ds(q * CHUNK, CHUNK)]], bufs[b], gsem)

  def wait_gather(b):
    pltpu.make_async_copy(
        table_sh.at[idx_v.at[0, pl.ds(0, CHUNK)]], bufs[b], gsem).wait()

  def start_write(b, j, q):
    pltpu.async_copy(
        bufs[b], out_hbm.at[j, pl.ds(col0 + q * CHUNK, CHUNK)], wsem)

  def drain_write(b):
    pltpu.make_async_copy(
        bufs[b], out_hbm.at[0, pl.ds(0, CHUNK)], wsem).wait()

  for q in range(3):
    start_gather(q, 0, q)

  def row_body(j, _):
    for q in range(4):
      wait_gather(q)
      start_write(q, j, q)
      if q == 0:
        pl.when(j >= 1)(lambda: drain_write(0))
      else:
        drain_write(q - 1)
      nq = (q + 3) % 4
      nj = j + (q + 3) // 4
      pl.when(nj < GROUPS)(lambda nj=nj, nq=nq: start_gather(nq, nj, nq))
    return 0

  lax.fori_loop(0, GROUPS, row_body, 0)
  drain_write(3)


def kernel(group_idx, table):
  idx_t = jnp.transpose(group_idx)
  mesh = plsc.VectorSubcoreMesh(core_axis_name="c", subcore_axis_name="s")
  run = functools.partial(
      pl.kernel,
      out_type=jax.ShapeDtypeStruct((GROUPS, DIM0, EMBED), jnp.float32),
      mesh=mesh,
      compiler_params=pltpu.CompilerParams(needs_layout_passes=False),
      scratch_types=[
          pltpu.VMEM((GROUPS, CPW), jnp.int32),
          pltpu.VMEM_SHARED((ROWS, EMBED), jnp.float32),
          pltpu.VMEM((CHUNK, EMBED), jnp.float32),
          pltpu.VMEM((CHUNK, EMBED), jnp.float32),
          pltpu.VMEM((CHUNK, EMBED), jnp.float32),
          pltpu.VMEM((CHUNK, EMBED), jnp.float32),
          pltpu.SemaphoreType.DMA,
          pltpu.SemaphoreType.DMA,
      ],
  )(_lookup)
  out_t = run(idx_t, table)
  return jnp.transpose(out_t, (1, 0, 2))
